# Initial kernel scaffold; baseline (speedup 1.0000x reference)
import jax
import jax.numpy as jnp
from jax.experimental import pallas as pl


def kernel(pc0, pc1, pc2, pc3, offset, W, b):
    raise NotImplementedError("write your pallas kernel here")


# speedup vs baseline: 8.8135x; 8.8135x over previous
import functools

import jax
import jax.numpy as jnp
from jax import lax
from jax.experimental import pallas as pl
from jax.experimental.pallas import tpu as pltpu

N = 32768
B = 16
GRID = 256.0
D_OUT = 128
ROWS_2D = N // 128

BLK = 2048


def _seg_pass_body(off_ref, p0, p1, p2, p3, m0_out, m1_out, bi_out):
    x = (p0[...] + p2[...]) * 0.5
    y = (p1[...] + p3[...]) * 0.5
    i = lax.broadcasted_iota(jnp.int32, (ROWS_2D, 128), 0)
    j = lax.broadcasted_iota(jnp.int32, (ROWS_2D, 128), 1)
    r = i * 128 + j
    seg = jnp.zeros((ROWS_2D, 128), jnp.int32)
    for k in range(B):
        seg = seg + (r >= off_ref[k]).astype(jnp.int32)
    bi_out[...] = seg.astype(jnp.float32)
    neg = jnp.float32(-1e30)
    m0 = jnp.zeros((ROWS_2D, 128), jnp.float32)
    m1 = jnp.zeros((ROWS_2D, 128), jnp.float32)
    for k in range(B):
        mask = seg == k
        mk0 = jnp.max(jnp.where(mask, x, neg))
        mk1 = jnp.max(jnp.where(mask, y, neg))
        m0 = jnp.where(mask, mk0, m0)
        m1 = jnp.where(mask, mk1, m1)
    m0_out[...] = m0
    m1_out[...] = m1


SUB = BLK // 128


def _dense_body(p0, p1, p2, p3, m0, m1, bi, w_ref, b_ref, out_ref):
    a0 = p0[...]
    a1 = p1[...]
    a2 = p2[...]
    a3 = p3[...]
    x = (a0 + a2) * 0.5
    y = (a1 + a3) * 0.5
    wdt = a2 - a0
    hgt = a3 - a1
    area = wdt * hgt
    xn = x / m0[...] * GRID
    yn = y / m1[...] * GRID
    feats = [bi[...], xn, yn, a0, a1, a2, a3, wdt, hgt, area]
    wmat = w_ref[...]
    bvec = b_ref[...]
    for i in range(SUB):
        ft = jnp.concatenate([f[i : i + 1, :] for f in feats], axis=0)
        o = lax.dot_general(
            ft, wmat, (((0,), (0,)), ((), ())),
            preferred_element_type=jnp.float32,
        )
        out_ref[pl.ds(i * 128, 128), :] = jnp.maximum(o + bvec, 0.0)


@jax.jit
def kernel(pc0, pc1, pc2, pc3, offset, W, b):
    pcs2d = [a.reshape(ROWS_2D, 128) for a in (pc0, pc1, pc2, pc3)]
    m0, m1, bi = pl.pallas_call(
        _seg_pass_body,
        grid_spec=pltpu.PrefetchScalarGridSpec(
            num_scalar_prefetch=1,
            grid=(),
            in_specs=[pl.BlockSpec((ROWS_2D, 128), lambda *_: (0, 0))] * 4,
            out_specs=[pl.BlockSpec((ROWS_2D, 128), lambda *_: (0, 0))] * 3,
        ),
        out_shape=[jax.ShapeDtypeStruct((ROWS_2D, 128), jnp.float32)] * 3,
    )(offset, *pcs2d)

    nblk = N // BLK
    out = pl.pallas_call(
        _dense_body,
        grid=(nblk,),
        in_specs=[pl.BlockSpec((SUB, 128), lambda i: (i, 0))] * 7
        + [
            pl.BlockSpec((10, D_OUT), lambda i: (0, 0)),
            pl.BlockSpec((1, D_OUT), lambda i: (0, 0)),
        ],
        out_specs=pl.BlockSpec((BLK, D_OUT), lambda i: (i, 0)),
        out_shape=jax.ShapeDtypeStruct((N, D_OUT), jnp.float32),
    )(*pcs2d, m0, m1, bi, W, b.reshape(1, D_OUT))
    return out


# speedup vs baseline: 10.8770x; 1.2341x over previous
import functools

import jax
import jax.numpy as jnp
from jax import lax
from jax.experimental import pallas as pl
from jax.experimental.pallas import tpu as pltpu

N = 32768
B = 16
GRID = 256.0
D_OUT = 128
ROWS_2D = N // 128

BLK = 2048
SUB = BLK // 128
NBLK = N // BLK


def _body(off_ref, p0, p1, p2, p3, w_ref, b_ref, out_ref, m0s, m1s):
    i = pl.program_id(0)

    @pl.when(i == 0)
    def _seg_maxes():
        x = (p0[...] + p2[...]) * 0.5
        y = (p1[...] + p3[...]) * 0.5
        ii = lax.broadcasted_iota(jnp.int32, (ROWS_2D, 128), 0)
        jj = lax.broadcasted_iota(jnp.int32, (ROWS_2D, 128), 1)
        r = ii * 128 + jj
        seg = jnp.zeros((ROWS_2D, 128), jnp.int32)
        for k in range(B):
            seg = seg + (r >= off_ref[k]).astype(jnp.int32)
        neg = jnp.float32(-1e30)
        for k in range(B):
            mask = seg == k
            m0s[k] = jnp.max(jnp.where(mask, x, neg))
            m1s[k] = jnp.max(jnp.where(mask, y, neg))

    a0 = p0[pl.ds(i * SUB, SUB), :]
    a1 = p1[pl.ds(i * SUB, SUB), :]
    a2 = p2[pl.ds(i * SUB, SUB), :]
    a3 = p3[pl.ds(i * SUB, SUB), :]
    x = (a0 + a2) * 0.5
    y = (a1 + a3) * 0.5
    wdt = a2 - a0
    hgt = a3 - a1
    area = wdt * hgt
    ii = lax.broadcasted_iota(jnp.int32, (SUB, 128), 0)
    jj = lax.broadcasted_iota(jnp.int32, (SUB, 128), 1)
    r = i * BLK + ii * 128 + jj
    seg = jnp.zeros((SUB, 128), jnp.int32)
    for k in range(B):
        seg = seg + (r >= off_ref[k]).astype(jnp.int32)
    m0v = jnp.zeros((SUB, 128), jnp.float32)
    m1v = jnp.zeros((SUB, 128), jnp.float32)
    for k in range(B):
        mask = seg == k
        m0v = jnp.where(mask, m0s[k], m0v)
        m1v = jnp.where(mask, m1s[k], m1v)
    xn = x / m0v * GRID
    yn = y / m1v * GRID
    bi = seg.astype(jnp.float32)
    feats = [bi, xn, yn, a0, a1, a2, a3, wdt, hgt, area]
    wmat = w_ref[...]
    bvec = b_ref[...]
    for g in range(SUB):
        ft = jnp.concatenate([f[g : g + 1, :] for f in feats], axis=0)
        o = lax.dot_general(
            ft, wmat, (((0,), (0,)), ((), ())),
            preferred_element_type=jnp.float32,
        )
        out_ref[pl.ds(g * 128, 128), :] = jnp.maximum(o + bvec, 0.0)


@jax.jit
def kernel(pc0, pc1, pc2, pc3, offset, W, b):
    pcs2d = [a.reshape(ROWS_2D, 128) for a in (pc0, pc1, pc2, pc3)]
    out = pl.pallas_call(
        _body,
        grid_spec=pltpu.PrefetchScalarGridSpec(
            num_scalar_prefetch=1,
            grid=(NBLK,),
            in_specs=[pl.BlockSpec((ROWS_2D, 128), lambda i, *_: (0, 0))] * 4
            + [
                pl.BlockSpec((10, D_OUT), lambda i, *_: (0, 0)),
                pl.BlockSpec((1, D_OUT), lambda i, *_: (0, 0)),
            ],
            out_specs=pl.BlockSpec((BLK, D_OUT), lambda i, *_: (i, 0)),
            scratch_shapes=[
                pltpu.SMEM((B,), jnp.float32),
                pltpu.SMEM((B,), jnp.float32),
            ],
        ),
        out_shape=jax.ShapeDtypeStruct((N, D_OUT), jnp.float32),
    )(offset, *pcs2d, W, b.reshape(1, D_OUT))
    return out


# speedup vs baseline: 13.8103x; 1.2697x over previous
import functools

import jax
import jax.numpy as jnp
from jax import lax
from jax.experimental import pallas as pl
from jax.experimental.pallas import tpu as pltpu

N = 32768
B = 16
GRID = 256.0
D_OUT = 128
ROWS_2D = N // 128

BLK = 4096
SUB = BLK // 128
NBLK = N // BLK


def _body(off_ref, p0, p1, p2, p3, w_ref, b_ref, out_ref, m0s, m1s):
    i = pl.program_id(0)

    @pl.when(i == 0)
    def _seg_maxes():
        x = (p0[...] + p2[...]) * 0.5
        y = (p1[...] + p3[...]) * 0.5
        ii = lax.broadcasted_iota(jnp.int32, (ROWS_2D, 128), 0)
        jj = lax.broadcasted_iota(jnp.int32, (ROWS_2D, 128), 1)
        r = ii * 128 + jj
        seg = jnp.zeros((ROWS_2D, 128), jnp.int32)
        for k in range(B):
            seg = seg + (r >= off_ref[k]).astype(jnp.int32)
        neg = jnp.float32(-1e30)
        for k in range(B):
            mask = seg == k
            m0s[k] = jnp.max(jnp.where(mask, x, neg))
            m1s[k] = jnp.max(jnp.where(mask, y, neg))

    a0 = p0[pl.ds(i * SUB, SUB), :]
    a1 = p1[pl.ds(i * SUB, SUB), :]
    a2 = p2[pl.ds(i * SUB, SUB), :]
    a3 = p3[pl.ds(i * SUB, SUB), :]
    x = (a0 + a2) * 0.5
    y = (a1 + a3) * 0.5
    wdt = a2 - a0
    hgt = a3 - a1
    area = wdt * hgt
    ii = lax.broadcasted_iota(jnp.int32, (SUB, 128), 0)
    jj = lax.broadcasted_iota(jnp.int32, (SUB, 128), 1)
    r = i * BLK + ii * 128 + jj
    seg = jnp.zeros((SUB, 128), jnp.int32)
    for k in range(B):
        seg = seg + (r >= off_ref[k]).astype(jnp.int32)
    m0v = jnp.zeros((SUB, 128), jnp.float32)
    m1v = jnp.zeros((SUB, 128), jnp.float32)
    for k in range(B):
        mask = seg == k
        m0v = jnp.where(mask, m0s[k], m0v)
        m1v = jnp.where(mask, m1s[k], m1v)
    xn = x / m0v * GRID
    yn = y / m1v * GRID
    bi = seg.astype(jnp.float32)
    feats = [bi, xn, yn, a0, a1, a2, a3, wdt, hgt, area]
    wmat = w_ref[...]
    bvec = b_ref[...]
    for g in range(SUB):
        ft = jnp.concatenate([f[g : g + 1, :] for f in feats], axis=0)
        o = lax.dot_general(
            ft, wmat, (((0,), (0,)), ((), ())),
            preferred_element_type=jnp.float32,
        )
        out_ref[pl.ds(g * 128, 128), :] = jnp.maximum(o + bvec, 0.0)


@jax.jit
def kernel(pc0, pc1, pc2, pc3, offset, W, b):
    pcs2d = [a.reshape(ROWS_2D, 128) for a in (pc0, pc1, pc2, pc3)]
    out = pl.pallas_call(
        _body,
        grid_spec=pltpu.PrefetchScalarGridSpec(
            num_scalar_prefetch=1,
            grid=(NBLK,),
            in_specs=[pl.BlockSpec((ROWS_2D, 128), lambda i, *_: (0, 0))] * 4
            + [
                pl.BlockSpec((10, D_OUT), lambda i, *_: (0, 0)),
                pl.BlockSpec((1, D_OUT), lambda i, *_: (0, 0)),
            ],
            out_specs=pl.BlockSpec((BLK, D_OUT), lambda i, *_: (i, 0)),
            scratch_shapes=[
                pltpu.SMEM((B,), jnp.float32),
                pltpu.SMEM((B,), jnp.float32),
            ],
        ),
        out_shape=jax.ShapeDtypeStruct((N, D_OUT), jnp.float32),
    )(offset, *pcs2d, W, b.reshape(1, D_OUT))
    return out


# speedup vs baseline: 15.4806x; 1.1209x over previous
import functools

import jax
import jax.numpy as jnp
from jax import lax
from jax.experimental import pallas as pl
from jax.experimental.pallas import tpu as pltpu

N = 32768
B = 16
GRID = 256.0
D_OUT = 128
ROWS_2D = N // 128

BLK = 8192
SUB = BLK // 128
NBLK = N // BLK


def _body(off_ref, p0, p1, p2, p3, w_ref, b_ref, out_ref, m0s, m1s):
    i = pl.program_id(0)

    @pl.when(i == 0)
    def _seg_maxes():
        x = (p0[...] + p2[...]) * 0.5
        y = (p1[...] + p3[...]) * 0.5
        ii = lax.broadcasted_iota(jnp.int32, (ROWS_2D, 128), 0)
        jj = lax.broadcasted_iota(jnp.int32, (ROWS_2D, 128), 1)
        r = ii * 128 + jj
        seg = jnp.zeros((ROWS_2D, 128), jnp.int32)
        for k in range(B):
            seg = seg + (r >= off_ref[k]).astype(jnp.int32)
        neg = jnp.float32(-1e30)
        for k in range(B):
            mask = seg == k
            m0s[k] = jnp.max(jnp.where(mask, x, neg))
            m1s[k] = jnp.max(jnp.where(mask, y, neg))

    a0 = p0[pl.ds(i * SUB, SUB), :]
    a1 = p1[pl.ds(i * SUB, SUB), :]
    a2 = p2[pl.ds(i * SUB, SUB), :]
    a3 = p3[pl.ds(i * SUB, SUB), :]
    x = (a0 + a2) * 0.5
    y = (a1 + a3) * 0.5
    wdt = a2 - a0
    hgt = a3 - a1
    area = wdt * hgt
    ii = lax.broadcasted_iota(jnp.int32, (SUB, 128), 0)
    jj = lax.broadcasted_iota(jnp.int32, (SUB, 128), 1)
    r = i * BLK + ii * 128 + jj
    seg = jnp.zeros((SUB, 128), jnp.int32)
    for k in range(B):
        seg = seg + (r >= off_ref[k]).astype(jnp.int32)
    m0v = jnp.zeros((SUB, 128), jnp.float32)
    m1v = jnp.zeros((SUB, 128), jnp.float32)
    for k in range(B):
        mask = seg == k
        m0v = jnp.where(mask, m0s[k], m0v)
        m1v = jnp.where(mask, m1s[k], m1v)
    xn = x / m0v * GRID
    yn = y / m1v * GRID
    bi = seg.astype(jnp.float32)
    feats = [bi, xn, yn, a0, a1, a2, a3, wdt, hgt, area]
    wmat = w_ref[...]
    bvec = b_ref[...]
    for g in range(SUB):
        ft = jnp.concatenate([f[g : g + 1, :] for f in feats], axis=0)
        o = lax.dot_general(
            ft, wmat, (((0,), (0,)), ((), ())),
            preferred_element_type=jnp.float32,
        )
        out_ref[pl.ds(g * 128, 128), :] = jnp.maximum(o + bvec, 0.0)


@jax.jit
def kernel(pc0, pc1, pc2, pc3, offset, W, b):
    pcs2d = [a.reshape(ROWS_2D, 128) for a in (pc0, pc1, pc2, pc3)]
    out = pl.pallas_call(
        _body,
        grid_spec=pltpu.PrefetchScalarGridSpec(
            num_scalar_prefetch=1,
            grid=(NBLK,),
            in_specs=[pl.BlockSpec((ROWS_2D, 128), lambda i, *_: (0, 0))] * 4
            + [
                pl.BlockSpec((10, D_OUT), lambda i, *_: (0, 0)),
                pl.BlockSpec((1, D_OUT), lambda i, *_: (0, 0)),
            ],
            out_specs=pl.BlockSpec((BLK, D_OUT), lambda i, *_: (i, 0)),
            scratch_shapes=[
                pltpu.SMEM((B,), jnp.float32),
                pltpu.SMEM((B,), jnp.float32),
            ],
        ),
        out_shape=jax.ShapeDtypeStruct((N, D_OUT), jnp.float32),
    )(offset, *pcs2d, W, b.reshape(1, D_OUT))
    return out
